# baseline (speedup 1.0000x reference)
import jax
import jax.numpy as jnp
from jax.experimental import pallas as pl
from jax.experimental.pallas import tpu as pltpu

N_STATES = 8
N_ACTIONS = 4
HIDDEN = 50
HID_P = 56
TILE_C = 65536


def _mlp_t_kernel(x_ref, w1t_ref, b1c_ref, w2t_ref, b2c_ref, o_ref):
    h = jnp.dot(w1t_ref[...], x_ref[...], preferred_element_type=jnp.float32)
    h = jnp.maximum(h + b1c_ref[...], 0.0)
    q = jnp.dot(w2t_ref[...], h, preferred_element_type=jnp.float32)
    o_ref[...] = q + b2c_ref[...]


def kernel(x, w1, b1, w2, b2):
    B = x.shape[0]

    w1t = jnp.zeros((HID_P, N_STATES), jnp.float32).at[:HIDDEN].set(w1.T)
    b1c = jnp.zeros((HID_P, 1), jnp.float32).at[:HIDDEN, 0].set(b1[0])
    w2t = jnp.zeros((N_ACTIONS, HID_P), jnp.float32).at[:, :HIDDEN].set(w2.T)
    b2c = b2.T

    b_pad = -(-B // TILE_C) * TILE_C
    x_p = x if b_pad == B else jnp.zeros((b_pad, N_STATES), jnp.float32).at[:B].set(x)
    x_t = jax.lax.optimization_barrier(x_p.T)

    flops = 2 * b_pad * (N_STATES * HID_P + HID_P * N_ACTIONS)
    bytes_accessed = 4 * b_pad * (N_STATES + N_ACTIONS) + 4 * (
        HID_P * N_STATES + HID_P + N_ACTIONS * HID_P + N_ACTIONS)

    out_t = pl.pallas_call(
        _mlp_t_kernel,
        out_shape=jax.ShapeDtypeStruct((N_ACTIONS, b_pad), jnp.float32),
        grid=(b_pad // TILE_C,),
        in_specs=[
            pl.BlockSpec((N_STATES, TILE_C), lambda i: (0, i)),
            pl.BlockSpec((HID_P, N_STATES), lambda i: (0, 0)),
            pl.BlockSpec((HID_P, 1), lambda i: (0, 0)),
            pl.BlockSpec((N_ACTIONS, HID_P), lambda i: (0, 0)),
            pl.BlockSpec((N_ACTIONS, 1), lambda i: (0, 0)),
        ],
        out_specs=pl.BlockSpec((N_ACTIONS, TILE_C), lambda i: (0, i)),
        compiler_params=pltpu.CompilerParams(
            dimension_semantics=("parallel",)),
        cost_estimate=pl.CostEstimate(flops=flops, transcendentals=0,
                                      bytes_accessed=bytes_accessed),
    )(x_t, w1t, b1c, w2t, b2c)

    return out_t.T[:B]
